# Initial kernel scaffold; baseline (speedup 1.0000x reference)
import functools

import jax
import jax.numpy as jnp
from jax import lax
from jax.experimental import pallas as pl
from jax.experimental.pallas import tpu as pltpu
from jax.experimental.pallas import tpu_sc as plsc

N_NODES = 100000
NUM_GRAPHS = 512
NC = 2
NS = 16
NW = NC * NS

CHUNK = 3136
LAST = N_NODES - (NW - 1) * CHUNK
WIN = 128
NWIN_FULL = CHUNK // WIN
TAIL_FULL = CHUNK - NWIN_FULL * WIN
NWIN_LAST = LAST // WIN
TAIL_LAST = LAST - NWIN_LAST * WIN


def _rsqrt(sq):
  ii = plsc.bitcast(sq, jnp.int32)
  ii = 0x5F3759DF - (ii >> 1)
  r = plsc.bitcast(ii, jnp.float32)
  hs = 0.5 * sq
  r = r * (1.5 - hs * r * r)
  r = r * (1.5 - hs * r * r)
  r = r * (1.5 - hs * r * r)
  return r


def _sc_body(dip_hbm, batch_hbm, zero_hbm, part_hbm,
             dip_v, out_v, idx_v, idx_t64, idx_t96, acc_sh):
  c = lax.axis_index("c")
  s = lax.axis_index("s")
  wid = c * NS + s
  is_last = wid == NW - 1
  base = wid * CHUNK

  @pl.when(s == 0)
  def _():
    pltpu.sync_copy(zero_hbm, acc_sh)
  plsc.subcore_barrier()

  @pl.when(jnp.logical_not(is_last))
  def _():
    pltpu.sync_copy(dip_hbm.at[pl.ds(base * 4, CHUNK * 4)], dip_v)

  @pl.when(is_last)
  def _():
    pltpu.sync_copy(dip_hbm.at[pl.ds(base * 4, LAST * 4)],
                    dip_v.at[pl.ds(0, LAST * 4)])

  n_groups = jnp.where(is_last, LAST // 16, CHUNK // 16)
  iota = lax.iota(jnp.int32, 16)
  col1 = jnp.full((16,), 1, jnp.int32)
  col2 = jnp.full((16,), 2, jnp.int32)

  def group(g, carry):
    b4 = g * 64
    idx = b4 + 4 * iota
    sv = plsc.load_gather(dip_v, [idx])
    xv = plsc.load_gather(dip_v, [idx + 1])
    yv = plsc.load_gather(dip_v, [idx + 2])
    zv = plsc.load_gather(dip_v, [idx + 3])
    sq = xv * xv + yv * yv + zv * zv
    f = sv * _rsqrt(sq)
    rows = g * 16 + iota
    plsc.store_scatter(out_v, [rows, col1 - 1], f * xv)
    plsc.store_scatter(out_v, [rows, col1], f * yv)
    plsc.store_scatter(out_v, [rows, col2], f * zv)
    return carry

  lax.fori_loop(0, n_groups, group, 0)

  def win(w, carry):
    off = w * WIN
    pltpu.sync_copy(batch_hbm.at[pl.ds(base + off, WIN)], idx_v)
    pltpu.sync_copy(out_v.at[pl.ds(off, WIN)], acc_sh.at[idx_v], add=True)
    return carry

  n_win = jnp.where(is_last, NWIN_LAST, NWIN_FULL)
  lax.fori_loop(0, n_win, win, 0)

  @pl.when(jnp.logical_not(is_last))
  def _():
    off = NWIN_FULL * WIN
    pltpu.sync_copy(batch_hbm.at[pl.ds(base + off, TAIL_FULL)], idx_t64)
    pltpu.sync_copy(out_v.at[pl.ds(off, TAIL_FULL)], acc_sh.at[idx_t64],
                    add=True)

  @pl.when(is_last)
  def _():
    off = NWIN_LAST * WIN
    pltpu.sync_copy(batch_hbm.at[pl.ds(base + off, TAIL_LAST)], idx_t96)
    pltpu.sync_copy(out_v.at[pl.ds(off, TAIL_LAST)], acc_sh.at[idx_t96],
                    add=True)

  plsc.subcore_barrier()

  @pl.when(s == 0)
  def _():
    pltpu.sync_copy(acc_sh, part_hbm.at[c])


def _sc_partials(dip_flat, batch_i32, zeros):
  mesh = plsc.VectorSubcoreMesh(core_axis_name="c", subcore_axis_name="s")
  f = pl.kernel(
      _sc_body,
      out_type=jax.ShapeDtypeStruct((NC, NUM_GRAPHS, 3), jnp.float32),
      mesh=mesh,
      scratch_types=[
          pltpu.VMEM((CHUNK * 4,), jnp.float32),
          pltpu.VMEM((CHUNK, 3), jnp.float32),
          pltpu.VMEM((WIN,), jnp.int32),
          pltpu.VMEM((TAIL_FULL,), jnp.int32),
          pltpu.VMEM((TAIL_LAST,), jnp.int32),
          pltpu.VMEM_SHARED((NUM_GRAPHS, 3), jnp.float32),
      ],
  )
  return f(dip_flat, batch_i32, zeros)


def _finish_body(part_ref, o_ref):
  p = part_ref[0] + part_ref[1]
  o_ref[...] = jnp.sqrt(jnp.sum(p * p, axis=-1, keepdims=True))


def kernel(dipole, batch):
  dip_flat = dipole.reshape(-1)
  batch_i32 = batch.astype(jnp.int32)
  zeros = jnp.zeros((NUM_GRAPHS, 3), jnp.float32)
  part = _sc_partials(dip_flat, batch_i32, zeros)
  return pl.pallas_call(
      _finish_body,
      out_shape=jax.ShapeDtypeStruct((NUM_GRAPHS, 1), jnp.float32),
  )(part)


# speedup vs baseline: 3.3581x; 3.3581x over previous
import functools

import jax
import jax.numpy as jnp
from jax import lax
from jax.experimental import pallas as pl
from jax.experimental.pallas import tpu as pltpu
from jax.experimental.pallas import tpu_sc as plsc

N_NODES = 100000
NUM_GRAPHS = 512
NC = 2
NS = 16
NW = NC * NS

CHUNK = 3136
LAST = N_NODES - (NW - 1) * CHUNK
WIN = 128
NWIN_FULL = CHUNK // WIN
TAIL_FULL = CHUNK - NWIN_FULL * WIN
NWIN_LAST = LAST // WIN
TAIL_LAST = LAST - NWIN_LAST * WIN


def _rsqrt(sq):
  ii = plsc.bitcast(sq, jnp.int32)
  ii = 0x5F3759DF - (ii >> 1)
  r = plsc.bitcast(ii, jnp.float32)
  hs = 0.5 * sq
  r = r * (1.5 - hs * r * r)
  r = r * (1.5 - hs * r * r)
  r = r * (1.5 - hs * r * r)
  return r


def _sc_body(dip_hbm, batch_hbm, zero_hbm, part_hbm,
             dip_v, out_v, idx_v, idx_t64, idx_t96, acc_sh):
  c = lax.axis_index("c")
  s = lax.axis_index("s")
  wid = c * NS + s
  is_last = wid == NW - 1
  base = wid * CHUNK

  @pl.when(s == 0)
  def _():
    pltpu.sync_copy(zero_hbm, acc_sh)
  plsc.subcore_barrier()

  @pl.when(jnp.logical_not(is_last))
  def _():
    pltpu.sync_copy(dip_hbm.at[pl.ds(base * 4, CHUNK * 4)], dip_v)

  @pl.when(is_last)
  def _():
    pltpu.sync_copy(dip_hbm.at[pl.ds(base * 4, LAST * 4)],
                    dip_v.at[pl.ds(0, LAST * 4)])

  n_groups = jnp.where(is_last, LAST // 16, CHUNK // 16)
  iota = lax.iota(jnp.int32, 16)
  col1 = jnp.full((16,), 1, jnp.int32)
  col2 = jnp.full((16,), 2, jnp.int32)

  def group(g, carry):
    b4 = g * 64
    idx = b4 + 4 * iota
    sv = plsc.load_gather(dip_v, [idx])
    xv = plsc.load_gather(dip_v, [idx + 1])
    yv = plsc.load_gather(dip_v, [idx + 2])
    zv = plsc.load_gather(dip_v, [idx + 3])
    sq = xv * xv + yv * yv + zv * zv
    f = sv * _rsqrt(sq)
    rows = g * 16 + iota
    plsc.store_scatter(out_v, [rows, col1 - 1], f * xv)
    plsc.store_scatter(out_v, [rows, col1], f * yv)
    plsc.store_scatter(out_v, [rows, col2], f * zv)
    return carry

  lax.fori_loop(0, n_groups, group, 0)

  def win(w, carry):
    off = w * WIN
    pltpu.sync_copy(batch_hbm.at[pl.ds(base + off, WIN)], idx_v)
    pltpu.sync_copy(out_v.at[pl.ds(off, WIN)], acc_sh.at[idx_v], add=True)
    return carry

  n_win = jnp.where(is_last, NWIN_LAST, NWIN_FULL)
  lax.fori_loop(0, n_win, win, 0)

  @pl.when(jnp.logical_not(is_last))
  def _():
    off = NWIN_FULL * WIN
    pltpu.sync_copy(batch_hbm.at[pl.ds(base + off, TAIL_FULL)], idx_t64)
    pltpu.sync_copy(out_v.at[pl.ds(off, TAIL_FULL)], acc_sh.at[idx_t64],
                    add=True)

  @pl.when(is_last)
  def _():
    off = NWIN_LAST * WIN
    pltpu.sync_copy(batch_hbm.at[pl.ds(base + off, TAIL_LAST)], idx_t96)
    pltpu.sync_copy(out_v.at[pl.ds(off, TAIL_LAST)], acc_sh.at[idx_t96],
                    add=True)

  plsc.subcore_barrier()

  @pl.when(s == 0)
  def _():
    pltpu.sync_copy(acc_sh, part_hbm.at[c])


def _sc_partials(dip_flat, batch_i32, zeros):
  mesh = plsc.VectorSubcoreMesh(core_axis_name="c", subcore_axis_name="s")
  f = pl.kernel(
      _sc_body,
      out_type=jax.ShapeDtypeStruct((NC, NUM_GRAPHS, 3), jnp.float32),
      mesh=mesh,
      compiler_params=pltpu.CompilerParams(
          needs_layout_passes=False, use_tc_tiling_on_sc=False),
      scratch_types=[
          pltpu.VMEM((CHUNK * 4,), jnp.float32),
          pltpu.VMEM((CHUNK, 3), jnp.float32),
          pltpu.VMEM((WIN,), jnp.int32),
          pltpu.VMEM((TAIL_FULL,), jnp.int32),
          pltpu.VMEM((TAIL_LAST,), jnp.int32),
          pltpu.VMEM_SHARED((NUM_GRAPHS, 3), jnp.float32),
      ],
  )
  return f(dip_flat, batch_i32, zeros)


def _finish_body(part_ref, o_ref):
  p = part_ref[0] + part_ref[1]
  o_ref[...] = jnp.sqrt(jnp.sum(p * p, axis=-1, keepdims=True))


def kernel(dipole, batch):
  dip_flat = dipole.reshape(-1)
  batch_i32 = batch.astype(jnp.int32)
  zeros = jnp.zeros((NUM_GRAPHS, 3), jnp.float32)
  part = _sc_partials(dip_flat, batch_i32, zeros)
  return pl.pallas_call(
      _finish_body,
      out_shape=jax.ShapeDtypeStruct((NUM_GRAPHS, 1), jnp.float32),
  )(part)


# speedup vs baseline: 3.7514x; 1.1171x over previous
import jax
import jax.numpy as jnp
from jax import lax
from jax.experimental import pallas as pl
from jax.experimental.pallas import tpu as pltpu
from jax.experimental.pallas import tpu_sc as plsc

N_NODES = 100000
NUM_GRAPHS = 512
NC = 2
NS = 16
NW = NC * NS

CHUNK = 3136
LAST = N_NODES - (NW - 1) * CHUNK


def _rsqrt(sq):
  ii = plsc.bitcast(sq, jnp.int32)
  ii = 0x5F3759DF - (ii >> 1)
  r = plsc.bitcast(ii, jnp.float32)
  hs = 0.5 * sq
  r = r * (1.5 - hs * r * r)
  r = r * (1.5 - hs * r * r)
  r = r * (1.5 - hs * r * r)
  return r


def _sc_body(dip_hbm, batch_hbm, zero_hbm, part_hbm,
             dip_v, out_v, idx_full, idx_last, acc_sh, sem_d, sem_i):
  c = lax.axis_index("c")
  s = lax.axis_index("s")
  wid = c * NS + s
  is_last = wid == NW - 1
  not_last = jnp.logical_not(is_last)
  base = wid * CHUNK

  def in_copies(n, idx_ref):
    dip_cp = pltpu.make_async_copy(
        dip_hbm.at[pl.ds(base * 4, n * 4)], dip_v.at[pl.ds(0, n * 4)], sem_d)
    idx_cp = pltpu.make_async_copy(
        batch_hbm.at[pl.ds(base, n)], idx_ref, sem_i)
    return dip_cp, idx_cp

  @pl.when(not_last)
  def _():
    for cp in in_copies(CHUNK, idx_full):
      cp.start()

  @pl.when(is_last)
  def _():
    for cp in in_copies(LAST, idx_last):
      cp.start()

  @pl.when(s == 0)
  def _():
    pltpu.sync_copy(zero_hbm, acc_sh)
  plsc.subcore_barrier()

  iota = lax.iota(jnp.int32, 16)
  col1 = jnp.full((16,), 1, jnp.int32)
  col2 = jnp.full((16,), 2, jnp.int32)

  def group(g, carry):
    idx = g * 64 + 4 * iota
    sv = plsc.load_gather(dip_v, [idx])
    xv = plsc.load_gather(dip_v, [idx + 1])
    yv = plsc.load_gather(dip_v, [idx + 2])
    zv = plsc.load_gather(dip_v, [idx + 3])
    sq = xv * xv + yv * yv + zv * zv
    f = sv * _rsqrt(sq)
    rows = g * 16 + iota
    plsc.store_scatter(out_v, [rows, col1 - 1], f * xv)
    plsc.store_scatter(out_v, [rows, col1], f * yv)
    plsc.store_scatter(out_v, [rows, col2], f * zv)
    return carry

  def tile_work(n, idx_ref):
    dip_cp, idx_cp = in_copies(n, idx_ref)
    dip_cp.wait()
    lax.fori_loop(0, n // 16, group, 0)
    idx_cp.wait()
    pltpu.sync_copy(out_v.at[pl.ds(0, n)], acc_sh.at[idx_ref], add=True)

  @pl.when(not_last)
  def _():
    tile_work(CHUNK, idx_full)

  @pl.when(is_last)
  def _():
    tile_work(LAST, idx_last)

  plsc.subcore_barrier()

  @pl.when(s == 0)
  def _():
    pltpu.sync_copy(acc_sh, part_hbm.at[c])


def _sc_partials(dip_flat, batch_i32, zeros):
  mesh = plsc.VectorSubcoreMesh(core_axis_name="c", subcore_axis_name="s")
  f = pl.kernel(
      _sc_body,
      out_type=jax.ShapeDtypeStruct((NC, NUM_GRAPHS, 3), jnp.float32),
      mesh=mesh,
      compiler_params=pltpu.CompilerParams(
          needs_layout_passes=False, use_tc_tiling_on_sc=False),
      scratch_types=[
          pltpu.VMEM((CHUNK * 4,), jnp.float32),
          pltpu.VMEM((CHUNK, 3), jnp.float32),
          pltpu.VMEM((CHUNK,), jnp.int32),
          pltpu.VMEM((LAST,), jnp.int32),
          pltpu.VMEM_SHARED((NUM_GRAPHS, 3), jnp.float32),
          pltpu.SemaphoreType.DMA,
          pltpu.SemaphoreType.DMA,
      ],
  )
  return f(dip_flat, batch_i32, zeros)


def _finish_body(part_ref, o_ref):
  p = part_ref[0] + part_ref[1]
  o_ref[...] = jnp.sqrt(jnp.sum(p * p, axis=-1, keepdims=True))


def kernel(dipole, batch):
  dip_flat = dipole.reshape(-1)
  batch_i32 = batch.astype(jnp.int32)
  zeros = jnp.zeros((NUM_GRAPHS, 3), jnp.float32)
  part = _sc_partials(dip_flat, batch_i32, zeros)
  return pl.pallas_call(
      _finish_body,
      out_shape=jax.ShapeDtypeStruct((NUM_GRAPHS, 1), jnp.float32),
  )(part)


# speedup vs baseline: 3.7565x; 1.0014x over previous
import jax
import jax.numpy as jnp
from jax import lax
from jax.experimental import pallas as pl
from jax.experimental.pallas import tpu as pltpu
from jax.experimental.pallas import tpu_sc as plsc

N_NODES = 100000
NUM_GRAPHS = 512
NC = 2
NS = 16
NW = NC * NS

CHUNK = 3136
LAST = N_NODES - (NW - 1) * CHUNK


def _rsqrt(sq):
  ii = plsc.bitcast(sq, jnp.int32)
  ii = 0x5F3759DF - (ii >> 1)
  r = plsc.bitcast(ii, jnp.float32)
  hs = 0.5 * sq
  r = r * (1.5 - hs * r * r)
  r = r * (1.5 - hs * r * r)
  r = r * (1.5 - hs * r * r)
  return r


def _sc_body(dip_hbm, batch_hbm, zero_hbm, part_hbm,
             dip_v, out_v, idx_full, idx_last, acc_sh, sem_d, sem_i):
  c = lax.axis_index("c")
  s = lax.axis_index("s")
  wid = c * NS + s
  is_last = wid == NW - 1
  not_last = jnp.logical_not(is_last)
  base = wid * CHUNK

  def in_copies(n, idx_ref):
    dip_cp = pltpu.make_async_copy(
        dip_hbm.at[pl.ds(base * 4 // 128, n * 4 // 128)],
        dip_v.at[pl.ds(0, n * 4 // 128)], sem_d)
    idx_cp = pltpu.make_async_copy(
        batch_hbm.at[pl.ds(base, n)], idx_ref, sem_i)
    return dip_cp, idx_cp

  @pl.when(not_last)
  def _():
    for cp in in_copies(CHUNK, idx_full):
      cp.start()

  @pl.when(is_last)
  def _():
    for cp in in_copies(LAST, idx_last):
      cp.start()

  @pl.when(s == 0)
  def _():
    pltpu.sync_copy(zero_hbm, acc_sh)
  plsc.subcore_barrier()

  iota = lax.iota(jnp.int32, 16)
  col0 = jnp.full((16,), 0, jnp.int32)
  col1 = jnp.full((16,), 1, jnp.int32)
  col2 = jnp.full((16,), 2, jnp.int32)
  col3 = jnp.full((16,), 3, jnp.int32)

  def group(g, carry):
    rows = g * 16 + iota
    flat = g * 64 + 4 * iota
    r128 = flat >> 7
    c128 = flat & 127
    sv = plsc.load_gather(dip_v, [r128, c128])
    xv = plsc.load_gather(dip_v, [r128, c128 + 1])
    yv = plsc.load_gather(dip_v, [r128, c128 + 2])
    zv = plsc.load_gather(dip_v, [r128, c128 + 3])
    sq = xv * xv + yv * yv + zv * zv
    f = sv * _rsqrt(sq)
    plsc.store_scatter(out_v, [rows, col0], f * xv)
    plsc.store_scatter(out_v, [rows, col1], f * yv)
    plsc.store_scatter(out_v, [rows, col2], f * zv)
    return carry

  def tile_work(n, idx_ref):
    dip_cp, idx_cp = in_copies(n, idx_ref)
    dip_cp.wait()
    lax.fori_loop(0, n // 16, group, 0)
    idx_cp.wait()
    pltpu.sync_copy(out_v.at[pl.ds(0, n)], acc_sh.at[idx_ref], add=True)

  @pl.when(not_last)
  def _():
    tile_work(CHUNK, idx_full)

  @pl.when(is_last)
  def _():
    tile_work(LAST, idx_last)

  plsc.subcore_barrier()

  @pl.when(s == 0)
  def _():
    pltpu.sync_copy(acc_sh, part_hbm.at[c])


def _sc_partials(dip, batch_i32, zeros):
  mesh = plsc.VectorSubcoreMesh(core_axis_name="c", subcore_axis_name="s")
  f = pl.kernel(
      _sc_body,
      out_type=jax.ShapeDtypeStruct((NC, NUM_GRAPHS, 3), jnp.float32),
      mesh=mesh,
      compiler_params=pltpu.CompilerParams(
          needs_layout_passes=False, use_tc_tiling_on_sc=False),
      scratch_types=[
          pltpu.VMEM((CHUNK * 4 // 128, 128), jnp.float32),
          pltpu.VMEM((CHUNK, 3), jnp.float32),
          pltpu.VMEM((CHUNK,), jnp.int32),
          pltpu.VMEM((LAST,), jnp.int32),
          pltpu.VMEM_SHARED((NUM_GRAPHS, 3), jnp.float32),
          pltpu.SemaphoreType.DMA,
          pltpu.SemaphoreType.DMA,
      ],
  )
  return f(dip, batch_i32, zeros)


def _finish_body(part_ref, o_ref):
  p = part_ref[0] + part_ref[1]
  o_ref[...] = jnp.sqrt(jnp.sum(p * p, axis=-1, keepdims=True))


def kernel(dipole, batch):
  dip128 = dipole.reshape(N_NODES * 4 // 128, 128)
  batch_i32 = batch.astype(jnp.int32)
  zeros = jnp.zeros((NUM_GRAPHS, 3), jnp.float32)
  part = _sc_partials(dip128, batch_i32, zeros)
  return pl.pallas_call(
      _finish_body,
      out_shape=jax.ShapeDtypeStruct((NUM_GRAPHS, 1), jnp.float32),
  )(part)


# speedup vs baseline: 8.9310x; 2.3775x over previous
import jax
import jax.numpy as jnp
from jax import lax
from jax.experimental import pallas as pl
from jax.experimental.pallas import tpu as pltpu
from jax.experimental.pallas import tpu_sc as plsc

N_NODES = 100000
NUM_GRAPHS = 512
NC = 2
NS = 16
NW = NC * NS

CHUNK = 3136
LAST = N_NODES - (NW - 1) * CHUNK


def _rsqrt(sq):
  ii = plsc.bitcast(sq, jnp.int32)
  ii = 0x5F3759DF - (ii >> 1)
  r = plsc.bitcast(ii, jnp.float32)
  hs = 0.5 * sq
  r = r * (1.5 - hs * r * r)
  r = r * (1.5 - hs * r * r)
  r = r * (1.5 - hs * r * r)
  return r


def _sc_body(s_hbm, x_hbm, y_hbm, z_hbm, batch_hbm, zero_hbm, part_hbm,
             s_v, x_v, y_v, z_v, out_v, idx_full, idx_last, acc_sh,
             sem_d, sem_i):
  c = lax.axis_index("c")
  s = lax.axis_index("s")
  wid = c * NS + s
  is_last = wid == NW - 1
  not_last = jnp.logical_not(is_last)
  base = wid * CHUNK

  def in_copies(n, idx_ref):
    cps = [
        pltpu.make_async_copy(
            hbm.at[pl.ds(base, n)], vm.at[pl.ds(0, n)], sem_d)
        for hbm, vm in ((s_hbm, s_v), (x_hbm, x_v), (y_hbm, y_v),
                        (z_hbm, z_v))
    ]
    cps.append(pltpu.make_async_copy(
        batch_hbm.at[pl.ds(base, n)], idx_ref, sem_i))
    return cps

  @pl.when(not_last)
  def _():
    for cp in in_copies(CHUNK, idx_full):
      cp.start()

  @pl.when(is_last)
  def _():
    for cp in in_copies(LAST, idx_last):
      cp.start()

  @pl.when(s == 0)
  def _():
    pltpu.sync_copy(zero_hbm, acc_sh)
  plsc.subcore_barrier()

  iota = lax.iota(jnp.int32, 16)
  col0 = jnp.full((16,), 0, jnp.int32)
  col1 = jnp.full((16,), 1, jnp.int32)
  col2 = jnp.full((16,), 2, jnp.int32)

  def group(g, carry):
    o = g * 16
    lanes = o + iota
    sv = plsc.load_gather(s_v, [lanes])
    xv = plsc.load_gather(x_v, [lanes])
    yv = plsc.load_gather(y_v, [lanes])
    zv = plsc.load_gather(z_v, [lanes])
    sq = xv * xv + yv * yv + zv * zv
    f = sv * _rsqrt(sq)
    rows = o + iota
    plsc.store_scatter(out_v, [rows, col0], f * xv)
    plsc.store_scatter(out_v, [rows, col1], f * yv)
    plsc.store_scatter(out_v, [rows, col2], f * zv)
    return carry

  def tile_work(n, idx_ref):
    cps = in_copies(n, idx_ref)
    for cp in cps[:4]:
      cp.wait()
    lax.fori_loop(0, n // 16, group, 0)
    cps[4].wait()
    pltpu.sync_copy(out_v.at[pl.ds(0, n)], acc_sh.at[idx_ref], add=True)

  @pl.when(not_last)
  def _():
    tile_work(CHUNK, idx_full)

  @pl.when(is_last)
  def _():
    tile_work(LAST, idx_last)

  plsc.subcore_barrier()

  @pl.when(s == 0)
  def _():
    pltpu.sync_copy(acc_sh, part_hbm.at[c])


def _sc_partials(cols, batch_i32, zeros):
  mesh = plsc.VectorSubcoreMesh(
      core_axis_name="c", subcore_axis_name="s", num_cores=NC,
      num_subcores=NS)
  f = pl.kernel(
      _sc_body,
      out_type=jax.ShapeDtypeStruct((NC, NUM_GRAPHS, 3), jnp.float32),
      mesh=mesh,
      compiler_params=pltpu.CompilerParams(
          needs_layout_passes=False, use_tc_tiling_on_sc=False),
      scratch_types=[
          pltpu.VMEM((CHUNK,), jnp.float32),
          pltpu.VMEM((CHUNK,), jnp.float32),
          pltpu.VMEM((CHUNK,), jnp.float32),
          pltpu.VMEM((CHUNK,), jnp.float32),
          pltpu.VMEM((CHUNK, 3), jnp.float32),
          pltpu.VMEM((CHUNK,), jnp.int32),
          pltpu.VMEM((LAST,), jnp.int32),
          pltpu.VMEM_SHARED((NUM_GRAPHS, 3), jnp.float32),
          pltpu.SemaphoreType.DMA,
          pltpu.SemaphoreType.DMA,
      ],
  )
  return f(*cols, batch_i32, zeros)


def _finish_body(part_ref, o_ref):
  p = part_ref[0] + part_ref[1]
  o_ref[...] = jnp.sqrt(jnp.sum(p * p, axis=-1, keepdims=True))


def kernel(dipole, batch):
  cols = [dipole[:, k] for k in range(4)]
  batch_i32 = batch.astype(jnp.int32)
  zeros = jnp.zeros((NUM_GRAPHS, 3), jnp.float32)
  part = _sc_partials(cols, batch_i32, zeros)
  return pl.pallas_call(
      _finish_body,
      out_shape=jax.ShapeDtypeStruct((NUM_GRAPHS, 1), jnp.float32),
  )(part)


# speedup vs baseline: 10.2481x; 1.1475x over previous
import jax
import jax.numpy as jnp
from jax import lax
from jax.experimental import pallas as pl
from jax.experimental.pallas import tpu as pltpu
from jax.experimental.pallas import tpu_sc as plsc

N_NODES = 100000
NUM_GRAPHS = 512
NC = 2
NS = 16
NW = NC * NS

CHUNK = 3136
LAST = N_NODES - (NW - 1) * CHUNK


def _rsqrt(sq):
  ii = plsc.bitcast(sq, jnp.int32)
  ii = 0x5F3759DF - (ii >> 1)
  r = plsc.bitcast(ii, jnp.float32)
  hs = 0.5 * sq
  r = r * (1.5 - hs * r * r)
  r = r * (1.5 - hs * r * r)
  r = r * (1.5 - hs * r * r)
  return r


def _sc_body(dip_t_hbm, batch_hbm, zero_hbm, part_hbm,
             s_v, x_v, y_v, z_v, out_v, idx_full, idx_last, acc_sh,
             sem_d, sem_i):
  c = lax.axis_index("c")
  s = lax.axis_index("s")
  wid = c * NS + s
  is_last = wid == NW - 1
  not_last = jnp.logical_not(is_last)
  base = wid * CHUNK

  def in_copies(n, idx_ref):
    cps = [
        pltpu.make_async_copy(
            dip_t_hbm.at[k, pl.ds(base, n)], vm.at[pl.ds(0, n)], sem_d)
        for k, vm in enumerate((s_v, x_v, y_v, z_v))
    ]
    cps.append(pltpu.make_async_copy(
        batch_hbm.at[pl.ds(base, n)], idx_ref, sem_i))
    return cps

  @pl.when(not_last)
  def _():
    for cp in in_copies(CHUNK, idx_full):
      cp.start()

  @pl.when(is_last)
  def _():
    for cp in in_copies(LAST, idx_last):
      cp.start()

  @pl.when(s == 0)
  def _():
    pltpu.sync_copy(zero_hbm, acc_sh)
  plsc.subcore_barrier()

  iota = lax.iota(jnp.int32, 16)
  col0 = jnp.full((16,), 0, jnp.int32)
  col1 = jnp.full((16,), 1, jnp.int32)
  col2 = jnp.full((16,), 2, jnp.int32)

  def group(g, carry):
    o = g * 16
    lanes = o + iota
    sv = plsc.load_gather(s_v, [lanes])
    xv = plsc.load_gather(x_v, [lanes])
    yv = plsc.load_gather(y_v, [lanes])
    zv = plsc.load_gather(z_v, [lanes])
    sq = xv * xv + yv * yv + zv * zv
    f = sv * _rsqrt(sq)
    rows = o + iota
    plsc.store_scatter(out_v, [rows, col0], f * xv)
    plsc.store_scatter(out_v, [rows, col1], f * yv)
    plsc.store_scatter(out_v, [rows, col2], f * zv)
    return carry

  def tile_work(n, idx_ref):
    cps = in_copies(n, idx_ref)
    for cp in cps[:4]:
      cp.wait()
    lax.fori_loop(0, n // 16, group, 0)
    cps[4].wait()
    pltpu.sync_copy(out_v.at[pl.ds(0, n)], acc_sh.at[idx_ref], add=True)

  @pl.when(not_last)
  def _():
    tile_work(CHUNK, idx_full)

  @pl.when(is_last)
  def _():
    tile_work(LAST, idx_last)

  plsc.subcore_barrier()

  @pl.when(s == 0)
  def _():
    pltpu.sync_copy(acc_sh, part_hbm.at[c])


def _sc_partials(dip_t, batch_i32, zeros):
  mesh = plsc.VectorSubcoreMesh(
      core_axis_name="c", subcore_axis_name="s", num_cores=NC,
      num_subcores=NS)
  f = pl.kernel(
      _sc_body,
      out_type=jax.ShapeDtypeStruct((NC, NUM_GRAPHS, 3), jnp.float32),
      mesh=mesh,
      compiler_params=pltpu.CompilerParams(
          needs_layout_passes=False, use_tc_tiling_on_sc=False),
      scratch_types=[
          pltpu.VMEM((CHUNK,), jnp.float32),
          pltpu.VMEM((CHUNK,), jnp.float32),
          pltpu.VMEM((CHUNK,), jnp.float32),
          pltpu.VMEM((CHUNK,), jnp.float32),
          pltpu.VMEM((CHUNK, 3), jnp.float32),
          pltpu.VMEM((CHUNK,), jnp.int32),
          pltpu.VMEM((LAST,), jnp.int32),
          pltpu.VMEM_SHARED((NUM_GRAPHS, 3), jnp.float32),
          pltpu.SemaphoreType.DMA,
          pltpu.SemaphoreType.DMA,
      ],
  )
  return f(dip_t, batch_i32, zeros)


def _finish_body(part_ref, o_ref):
  p = part_ref[0] + part_ref[1]
  o_ref[...] = jnp.sqrt(jnp.sum(p * p, axis=-1, keepdims=True))


def kernel(dipole, batch):
  dip_t = dipole.T
  batch_i32 = batch.astype(jnp.int32)
  zeros = jnp.zeros((NUM_GRAPHS, 3), jnp.float32)
  part = _sc_partials(dip_t, batch_i32, zeros)
  return pl.pallas_call(
      _finish_body,
      out_shape=jax.ShapeDtypeStruct((NUM_GRAPHS, 1), jnp.float32),
  )(part)
